# baseline (speedup 1.0000x reference)
import functools

import jax
import jax.numpy as jnp
from jax import lax
from jax.experimental import pallas as pl
from jax.experimental.pallas import tpu as pltpu
from jax.experimental.pallas import tpu_sc as plsc

N = 256
FEAT = 128
LATENT = 256
NF = 5
VTOT = 48
BI = 4


def _sc_gather(tables, ids):
    info = plsc.get_sparse_core_info()
    nw = info.num_cores * info.num_subcores
    rpw = N // nw
    mesh = plsc.VectorSubcoreMesh(core_axis_name="c", subcore_axis_name="s")

    @functools.partial(
        pl.kernel,
        mesh=mesh,
        out_type=jax.ShapeDtypeStruct((NF, N, FEAT), jnp.float32),
        scratch_types=[
            [pltpu.VMEM((rpw,), jnp.int32) for _ in range(NF)],
            [pltpu.VMEM((rpw, FEAT), jnp.float32) for _ in range(NF)],
            pltpu.SemaphoreType.DMA,
        ],
    )
    def gather_kernel(t0, t1, t2, t3, t4, i0, i1, i2, i3, i4, out_hbm,
                      idx_vs, rows_vs, sem):
        tabs = (t0, t1, t2, t3, t4)
        idhs = (i0, i1, i2, i3, i4)
        wid = lax.axis_index("s") * info.num_cores + lax.axis_index("c")
        base = wid * rpw
        for f in range(NF):
            pltpu.sync_copy(idhs[f].at[pl.ds(base, rpw)], idx_vs[f])
        gathers = [pltpu.async_copy(tabs[f].at[idx_vs[f]], rows_vs[f], sem)
                   for f in range(NF)]
        for g in gathers:
            g.wait()
        for f in range(NF):
            pltpu.sync_copy(rows_vs[f], out_hbm.at[f, pl.ds(base, rpw)])

    return gather_kernel(*tables, *ids)


def _dot_t(x, w):
    return lax.dot_general(x, w, (((1,), (1,)), ((), ())),
                           preferred_element_type=jnp.float32)


def _tc_body(embs_ref, w0, w1, w2, w3, w4, b0, b1, b2, b3, b4,
             o0, o1, o2, o3, o4, a_ref, ee_ref, w23_ref):
    i = pl.program_id(0)
    ws = (w0, w1, w2, w3, w4)
    bs = (b0, b1, b2, b3, b4)

    @pl.when(i == 0)
    def _():
        for f in range(NF):
            e = embs_ref[f]
            a_ref[f] = _dot_t(e, ws[f][:, 0:FEAT]) + bs[f][...]
            ee_ref[f] = jnp.concatenate([e, e], axis=1).astype(jnp.bfloat16)
            w23_ref[f] = jnp.concatenate(
                [ws[f][:, 2 * FEAT:3 * FEAT],
                 ws[f][:, FEAT:2 * FEAT]], axis=1).astype(jnp.bfloat16)

    outs = (o0, o1, o2, o3, o4)
    ones = jnp.ones((1, FEAT), jnp.bfloat16)
    for f in range(NF):
        for ii in range(BI):
            r = i * BI + ii
            row = embs_ref[f, pl.ds(r, 1), :].astype(jnp.bfloat16)
            rowext = jnp.concatenate([row, ones], axis=1)
            lhs = ee_ref[f] * rowext
            m = _dot_t(lhs, w23_ref[f])
            a_row = a_ref[f, pl.ds(r, 1), :]
            outs[f][pl.ds(ii * N, N), :] = jnp.maximum(m + a_row, 0.0)


def _tc_encode(embs_all, ws, bs, interpret=False):
    out_sd = jax.ShapeDtypeStruct((N * N, LATENT), jnp.float32)
    full = lambda shape: pl.BlockSpec(shape, lambda i: tuple(0 for _ in shape))
    return pl.pallas_call(
        _tc_body,
        grid=(N // BI,),
        in_specs=[full((NF, N, FEAT))]
        + [full((LATENT, 3 * FEAT))] * NF
        + [full((1, LATENT))] * NF,
        out_specs=[pl.BlockSpec((BI * N, LATENT), lambda i: (i, 0))] * NF,
        out_shape=[out_sd] * NF,
        scratch_shapes=[
            pltpu.VMEM((NF, N, LATENT), jnp.float32),
            pltpu.VMEM((NF, N, 2 * FEAT), jnp.bfloat16),
            pltpu.VMEM((NF, LATENT, 2 * FEAT), jnp.bfloat16),
        ],
        compiler_params=pltpu.CompilerParams(
            dimension_semantics=("arbitrary",),
        ),
        interpret=interpret,
    )(embs_all, *ws, *bs)


def kernel(typ_ids, typ_table, typ_W, typ_b,
           pol_ids, pol_table, pol_W, pol_b,
           mod_ids, mod_table, mod_W, mod_b,
           gen_ids, gen_table, gen_W, gen_b,
           ten_ids, ten_table, ten_W, ten_b):
    tables = (typ_table, pol_table, mod_table, gen_table, ten_table)
    ids = (typ_ids, pol_ids, mod_ids, gen_ids, ten_ids)

    embs_all = _sc_gather(tables, ids)

    ws = (typ_W, pol_W, mod_W, gen_W, ten_W)
    bs = tuple(b.reshape(1, LATENT) for b in (typ_b, pol_b, mod_b, gen_b, ten_b))

    return tuple(_tc_encode(embs_all, ws, bs))


# speedup vs baseline: 1.0024x; 1.0024x over previous
import functools

import jax
import jax.numpy as jnp
from jax import lax
from jax.experimental import pallas as pl
from jax.experimental.pallas import tpu as pltpu
from jax.experimental.pallas import tpu_sc as plsc

N = 256
FEAT = 128
LATENT = 256
NF = 5
VTOT = 48
BI = 4


def _sc_gather(tables, ids):
    info = plsc.get_sparse_core_info()
    nw = info.num_cores * info.num_subcores
    rpw = N // nw
    mesh = plsc.VectorSubcoreMesh(core_axis_name="c", subcore_axis_name="s")

    @functools.partial(
        pl.kernel,
        mesh=mesh,
        out_type=jax.ShapeDtypeStruct((NF, N, FEAT), jnp.float32),
        scratch_types=[
            [pltpu.VMEM((rpw,), jnp.int32) for _ in range(NF)],
            [pltpu.VMEM((rpw, FEAT), jnp.float32) for _ in range(NF)],
            pltpu.SemaphoreType.DMA,
        ],
    )
    def gather_kernel(t0, t1, t2, t3, t4, i0, i1, i2, i3, i4, out_hbm,
                      idx_vs, rows_vs, sem):
        tabs = (t0, t1, t2, t3, t4)
        idhs = (i0, i1, i2, i3, i4)
        wid = lax.axis_index("s") * info.num_cores + lax.axis_index("c")
        base = wid * rpw
        loads = [pltpu.async_copy(idhs[f].at[pl.ds(base, rpw)], idx_vs[f], sem)
                 for f in range(NF)]
        for c in loads:
            c.wait()
        gathers = [pltpu.async_copy(tabs[f].at[idx_vs[f]], rows_vs[f], sem)
                   for f in range(NF)]
        for c in gathers:
            c.wait()
        stores = [pltpu.async_copy(rows_vs[f], out_hbm.at[f, pl.ds(base, rpw)],
                                   sem)
                  for f in range(NF)]
        for c in stores:
            c.wait()

    return gather_kernel(*tables, *ids)


def _dot_t(x, w):
    return lax.dot_general(x, w, (((1,), (1,)), ((), ())),
                           preferred_element_type=jnp.float32)


def _tc_body(embs_ref, w0, w1, w2, w3, w4, b0, b1, b2, b3, b4,
             o0, o1, o2, o3, o4, a_ref, ee_ref, w23_ref):
    i = pl.program_id(0)
    ws = (w0, w1, w2, w3, w4)
    bs = (b0, b1, b2, b3, b4)

    @pl.when(i == 0)
    def _():
        for f in range(NF):
            e = embs_ref[f]
            a_ref[f] = _dot_t(e, ws[f][:, 0:FEAT]) + bs[f][...]
            ee_ref[f] = jnp.concatenate([e, e], axis=1).astype(jnp.bfloat16)
            w23_ref[f] = jnp.concatenate(
                [ws[f][:, 2 * FEAT:3 * FEAT],
                 ws[f][:, FEAT:2 * FEAT]], axis=1).astype(jnp.bfloat16)

    outs = (o0, o1, o2, o3, o4)
    ones = jnp.ones((1, FEAT), jnp.bfloat16)
    for f in range(NF):
        for ii in range(BI):
            r = i * BI + ii
            row = embs_ref[f, pl.ds(r, 1), :].astype(jnp.bfloat16)
            rowext = jnp.concatenate([row, ones], axis=1)
            lhs = ee_ref[f] * rowext
            m = _dot_t(lhs, w23_ref[f])
            a_row = a_ref[f, pl.ds(r, 1), :]
            outs[f][pl.ds(ii * N, N), :] = jnp.maximum(m + a_row, 0.0)


def _tc_encode(embs_all, ws, bs, interpret=False):
    out_sd = jax.ShapeDtypeStruct((N * N, LATENT), jnp.float32)
    full = lambda shape: pl.BlockSpec(shape, lambda i: tuple(0 for _ in shape))
    return pl.pallas_call(
        _tc_body,
        grid=(N // BI,),
        in_specs=[full((NF, N, FEAT))]
        + [full((LATENT, 3 * FEAT))] * NF
        + [full((1, LATENT))] * NF,
        out_specs=[pl.BlockSpec((BI * N, LATENT), lambda i: (i, 0))] * NF,
        out_shape=[out_sd] * NF,
        scratch_shapes=[
            pltpu.VMEM((NF, N, LATENT), jnp.float32),
            pltpu.VMEM((NF, N, 2 * FEAT), jnp.bfloat16),
            pltpu.VMEM((NF, LATENT, 2 * FEAT), jnp.bfloat16),
        ],
        compiler_params=pltpu.CompilerParams(
            dimension_semantics=("arbitrary",),
        ),
        interpret=interpret,
    )(embs_all, *ws, *bs)


def kernel(typ_ids, typ_table, typ_W, typ_b,
           pol_ids, pol_table, pol_W, pol_b,
           mod_ids, mod_table, mod_W, mod_b,
           gen_ids, gen_table, gen_W, gen_b,
           ten_ids, ten_table, ten_W, ten_b):
    tables = (typ_table, pol_table, mod_table, gen_table, ten_table)
    ids = (typ_ids, pol_ids, mod_ids, gen_ids, ten_ids)

    embs_all = _sc_gather(tables, ids)

    ws = (typ_W, pol_W, mod_W, gen_W, ten_W)
    bs = tuple(b.reshape(1, LATENT) for b in (typ_b, pol_b, mod_b, gen_b, ten_b))

    return tuple(_tc_encode(embs_all, ws, bs))


# speedup vs baseline: 1.0446x; 1.0421x over previous
import functools

import jax
import jax.numpy as jnp
from jax import lax
from jax.experimental import pallas as pl
from jax.experimental.pallas import tpu as pltpu
from jax.experimental.pallas import tpu_sc as plsc

N = 256
FEAT = 128
LATENT = 256
NF = 5
VTOT = 48
BI = 4


def _sc_gather(tables, ids):
    info = plsc.get_sparse_core_info()
    nw = info.num_subcores
    rpw = N // nw
    mesh = plsc.VectorSubcoreMesh(core_axis_name="c", subcore_axis_name="s",
                                  num_cores=1)

    @functools.partial(
        pl.kernel,
        mesh=mesh,
        out_type=jax.ShapeDtypeStruct((NF, N, FEAT), jnp.float32),
        scratch_types=[
            [pltpu.VMEM((rpw,), jnp.int32) for _ in range(NF)],
            [pltpu.VMEM((rpw, FEAT), jnp.float32) for _ in range(NF)],
            pltpu.SemaphoreType.DMA,
        ],
    )
    def gather_kernel(t0, t1, t2, t3, t4, i0, i1, i2, i3, i4, out_hbm,
                      idx_vs, rows_vs, sem):
        tabs = (t0, t1, t2, t3, t4)
        idhs = (i0, i1, i2, i3, i4)
        wid = lax.axis_index("s")
        base = wid * rpw
        loads = [pltpu.async_copy(idhs[f].at[pl.ds(base, rpw)], idx_vs[f], sem)
                 for f in range(NF)]
        for c in loads:
            c.wait()
        gathers = [pltpu.async_copy(tabs[f].at[idx_vs[f]], rows_vs[f], sem)
                   for f in range(NF)]
        for c in gathers:
            c.wait()
        stores = [pltpu.async_copy(rows_vs[f], out_hbm.at[f, pl.ds(base, rpw)],
                                   sem)
                  for f in range(NF)]
        for c in stores:
            c.wait()

    return gather_kernel(*tables, *ids)


def _dot_t(x, w):
    return lax.dot_general(x, w, (((1,), (1,)), ((), ())),
                           preferred_element_type=jnp.float32)


def _tc_body(embs_ref, w0, w1, w2, w3, w4, b0, b1, b2, b3, b4,
             o0, o1, o2, o3, o4, a_ref, ee_ref, w23_ref):
    i = pl.program_id(0)
    ws = (w0, w1, w2, w3, w4)
    bs = (b0, b1, b2, b3, b4)

    @pl.when(i == 0)
    def _():
        for f in range(NF):
            e = embs_ref[f]
            a_ref[f] = _dot_t(e, ws[f][:, 0:FEAT]) + bs[f][...]
            ee_ref[f] = jnp.concatenate([e, e], axis=1).astype(jnp.bfloat16)
            w23_ref[f] = jnp.concatenate(
                [ws[f][:, 2 * FEAT:3 * FEAT],
                 ws[f][:, FEAT:2 * FEAT]], axis=1).astype(jnp.bfloat16)

    outs = (o0, o1, o2, o3, o4)
    ones = jnp.ones((1, FEAT), jnp.bfloat16)
    for f in range(NF):
        for ii in range(BI):
            r = i * BI + ii
            row = embs_ref[f, pl.ds(r, 1), :].astype(jnp.bfloat16)
            rowext = jnp.concatenate([row, ones], axis=1)
            lhs = ee_ref[f] * rowext
            m = _dot_t(lhs, w23_ref[f])
            a_row = a_ref[f, pl.ds(r, 1), :]
            outs[f][pl.ds(ii * N, N), :] = jnp.maximum(m + a_row, 0.0)


def _tc_encode(embs_all, ws, bs, interpret=False):
    out_sd = jax.ShapeDtypeStruct((N * N, LATENT), jnp.float32)
    full = lambda shape: pl.BlockSpec(shape, lambda i: tuple(0 for _ in shape))
    return pl.pallas_call(
        _tc_body,
        grid=(N // BI,),
        in_specs=[full((NF, N, FEAT))]
        + [full((LATENT, 3 * FEAT))] * NF
        + [full((1, LATENT))] * NF,
        out_specs=[pl.BlockSpec((BI * N, LATENT), lambda i: (i, 0))] * NF,
        out_shape=[out_sd] * NF,
        scratch_shapes=[
            pltpu.VMEM((NF, N, LATENT), jnp.float32),
            pltpu.VMEM((NF, N, 2 * FEAT), jnp.bfloat16),
            pltpu.VMEM((NF, LATENT, 2 * FEAT), jnp.bfloat16),
        ],
        compiler_params=pltpu.CompilerParams(
            dimension_semantics=("arbitrary",),
        ),
        interpret=interpret,
    )(embs_all, *ws, *bs)


def kernel(typ_ids, typ_table, typ_W, typ_b,
           pol_ids, pol_table, pol_W, pol_b,
           mod_ids, mod_table, mod_W, mod_b,
           gen_ids, gen_table, gen_W, gen_b,
           ten_ids, ten_table, ten_W, ten_b):
    tables = (typ_table, pol_table, mod_table, gen_table, ten_table)
    ids = (typ_ids, pol_ids, mod_ids, gen_ids, ten_ids)

    embs_all = _sc_gather(tables, ids)

    ws = (typ_W, pol_W, mod_W, gen_W, ten_W)
    bs = tuple(b.reshape(1, LATENT) for b in (typ_b, pol_b, mod_b, gen_b, ten_b))

    return tuple(_tc_encode(embs_all, ws, bs))
